# baseline (speedup 1.0000x reference)
import functools

import jax
import jax.numpy as jnp
from jax import lax
from jax.experimental import pallas as pl
from jax.experimental.pallas import tpu as pltpu
from jax.experimental.pallas import tpu_sc as plsc

N_BEHAVIORS = 8
_LB = 8
_NW = 32
_LANES = 16


def _sc_mask_kernel(L, B):
    CW = B // _NW

    mesh = plsc.VectorSubcoreMesh(core_axis_name="c", subcore_axis_name="s")

    @functools.partial(
        pl.kernel,
        mesh=mesh,
        out_type=[
            jax.ShapeDtypeStruct((L, B), jnp.float32),
            jax.ShapeDtypeStruct((L, B), jnp.float32),
        ],
        scratch_types=[
            pltpu.VMEM((L, CW), jnp.int32),
            pltpu.VMEM((L, CW), jnp.float32),
            pltpu.VMEM((L, CW), jnp.float32),
            pltpu.VMEM((L, CW), jnp.float32),
            pltpu.VMEM((N_BEHAVIORS, _LANES), jnp.float32),
            pltpu.VMEM((L, _LANES), jnp.float32),
        ],
    )
    def k(pb_hbm, t_hbm, beh_hbm, pad_hbm, m_hbm, neg_hbm,
          beh_v, pad_v, m_v, neg_v, pb_v, t_v):
        wid = lax.axis_index("c") * (_NW // 2) + lax.axis_index("s")
        col0 = wid * CW
        pltpu.sync_copy(pb_hbm, pb_v)
        pltpu.sync_copy(t_hbm, t_v)
        pltpu.sync_copy(beh_hbm.at[:, pl.ds(col0, CW)], beh_v)
        pltpu.sync_copy(pad_hbm.at[:, pl.ds(col0, CW)], pad_v)

        def row(l, _):
            tl = t_v[l, :]

            def col(j, _):
                sl = pl.ds(j * _LANES, _LANES)
                idx = jnp.maximum(beh_v[l, sl] - 1, 0)
                pref = jnp.zeros((_LANES,), jnp.float32)
                for k in range(N_BEHAVIORS):
                    pref = jnp.where(idx == k, pb_v[k, :], pref)
                pad = pad_v[l, sl]
                m = jnp.where(pref - tl > 0.0, pad, jnp.zeros_like(pad))
                m_v[l, sl] = m
                neg_v[l, sl] = (1.0 - m) * pad
                return 0

            return lax.fori_loop(0, CW // _LANES, col, 0)

        lax.fori_loop(0, L, row, 0)
        pltpu.sync_copy(m_v, m_hbm.at[:, pl.ds(col0, CW)])
        pltpu.sync_copy(neg_v, neg_hbm.at[:, pl.ds(col0, CW)])

    return k


def _tc_body(m_ref, neg_ref, s_ref, hp_ref, hn_ref):
    s = s_ref[...]
    hp_ref[...] = s * m_ref[...][:, None, :]
    hn_ref[...] = s * neg_ref[...][:, None, :]


def kernel(S, behavior_seq, padding_mask, lambda_raw, threshold):
    B, L, D = S.shape
    lam = jax.nn.softplus(lambda_raw) + 1e-06
    log_pmf = -lam + lam * jnp.log(lam) - jax.lax.lgamma(lam + 1.0)
    p_b1 = jnp.zeros((_LANES,), jnp.float32).at[:N_BEHAVIORS].set(
        jnp.exp(log_pmf) + 1.0)
    p_b = jnp.tile(p_b1[:N_BEHAVIORS, None], (1, _LANES))
    t = jnp.tile(jax.nn.sigmoid(threshold[:L])[:, None], (1, _LANES))
    St = S.transpose(1, 2, 0)
    behT = behavior_seq.T
    padT = padding_mask.T

    m, neg = _sc_mask_kernel(L, B)(p_b, t, behT, padT)

    grid = (L // _LB,)
    hp, hn = pl.pallas_call(
        _tc_body,
        grid=grid,
        in_specs=[
            pl.BlockSpec((_LB, B), lambda i: (i, 0)),
            pl.BlockSpec((_LB, B), lambda i: (i, 0)),
            pl.BlockSpec((_LB, D, B), lambda i: (i, 0, 0)),
        ],
        out_specs=[
            pl.BlockSpec((_LB, D, B), lambda i: (i, 0, 0)),
            pl.BlockSpec((_LB, D, B), lambda i: (i, 0, 0)),
        ],
        out_shape=[
            jax.ShapeDtypeStruct((L, D, B), jnp.float32),
            jax.ShapeDtypeStruct((L, D, B), jnp.float32),
        ],
    )(m, neg, St)
    return (hp.transpose(2, 0, 1), hn.transpose(2, 0, 1))


# speedup vs baseline: 1.2178x; 1.2178x over previous
import jax
import jax.numpy as jnp
from jax.experimental import pallas as pl

N_BEHAVIORS = 8
_LB = 8


def _body(pb_ref, t_ref, beh_ref, pad_ref, s_ref, hp_ref, hn_ref):
    lb, bb = beh_ref.shape
    idx = jnp.maximum(beh_ref[...] - 1, 0)
    pref = jnp.zeros((lb, bb), jnp.float32)
    for k in range(N_BEHAVIORS):
        pref = pref + jnp.where(idx == k, pb_ref[0, k], 0.0)
    pad = pad_ref[...]
    m = jnp.where(pref - t_ref[...] > 0.0, pad, 0.0)
    neg = (1.0 - m) * pad
    s = s_ref[...]
    hp_ref[...] = s * m[:, None, :]
    hn_ref[...] = s * neg[:, None, :]


def kernel(S, behavior_seq, padding_mask, lambda_raw, threshold):
    B, L, D = S.shape
    lam = jax.nn.softplus(lambda_raw) + 1e-06
    log_pmf = -lam + lam * jnp.log(lam) - jax.lax.lgamma(lam + 1.0)
    p_b = (jnp.exp(log_pmf) + 1.0).reshape(1, N_BEHAVIORS)
    t = jax.nn.sigmoid(threshold[:L]).reshape(L, 1)
    St = S.transpose(1, 2, 0)
    behT = behavior_seq.T
    padT = padding_mask.T

    grid = (L // _LB, 2)
    hp, hn = pl.pallas_call(
        _body,
        grid=grid,
        in_specs=[
            pl.BlockSpec((1, N_BEHAVIORS), lambda i, j: (0, 0)),
            pl.BlockSpec((_LB, 1), lambda i, j: (i, 0)),
            pl.BlockSpec((_LB, B), lambda i, j: (i, 0)),
            pl.BlockSpec((_LB, B), lambda i, j: (i, 0)),
            pl.BlockSpec((_LB, D // 2, B), lambda i, j: (i, j, 0)),
        ],
        out_specs=[
            pl.BlockSpec((_LB, D // 2, B), lambda i, j: (i, j, 0)),
            pl.BlockSpec((_LB, D // 2, B), lambda i, j: (i, j, 0)),
        ],
        out_shape=[
            jax.ShapeDtypeStruct((L, D, B), jnp.float32),
            jax.ShapeDtypeStruct((L, D, B), jnp.float32),
        ],
    )(p_b, t, behT, padT, St)
    return (hp.transpose(2, 0, 1), hn.transpose(2, 0, 1))


# speedup vs baseline: 1.2543x; 1.0300x over previous
import jax
import jax.numpy as jnp
from jax.experimental import pallas as pl

N_BEHAVIORS = 8
_LB = 8


def _body(pb_ref, t_ref, beh_ref, pad_ref, s_ref, hp_ref, hn_ref):
    lb, bb = beh_ref.shape
    idx = jnp.maximum(beh_ref[...] - 1, 0)
    pref = jnp.zeros((lb, bb), jnp.float32)
    for k in range(N_BEHAVIORS):
        pref = pref + jnp.where(idx == k, pb_ref[0, k], 0.0)
    pad = pad_ref[...]
    m = jnp.where(pref - t_ref[...] > 0.0, pad, 0.0)
    neg = (1.0 - m) * pad
    s = s_ref[...]
    hp_ref[...] = s * m[:, None, :]
    hn_ref[...] = s * neg[:, None, :]


def kernel(S, behavior_seq, padding_mask, lambda_raw, threshold):
    B, L, D = S.shape
    lam = jax.nn.softplus(lambda_raw) + 1e-06
    log_pmf = -lam + lam * jnp.log(lam) - jax.lax.lgamma(lam + 1.0)
    p_b = (jnp.exp(log_pmf) + 1.0).reshape(1, N_BEHAVIORS)
    t = jax.nn.sigmoid(threshold[:L]).reshape(L, 1)
    St = S.transpose(1, 2, 0)
    behT = behavior_seq.T
    padT = padding_mask.T

    grid = (L // _LB,)
    hp, hn = pl.pallas_call(
        _body,
        grid=grid,
        in_specs=[
            pl.BlockSpec((1, N_BEHAVIORS), lambda i: (0, 0)),
            pl.BlockSpec((_LB, 1), lambda i: (i, 0)),
            pl.BlockSpec((_LB, B), lambda i: (i, 0)),
            pl.BlockSpec((_LB, B), lambda i: (i, 0)),
            pl.BlockSpec((_LB, D, B), lambda i: (i, 0, 0)),
        ],
        out_specs=[
            pl.BlockSpec((_LB, D, B), lambda i: (i, 0, 0)),
            pl.BlockSpec((_LB, D, B), lambda i: (i, 0, 0)),
        ],
        out_shape=[
            jax.ShapeDtypeStruct((L, D, B), jnp.float32),
            jax.ShapeDtypeStruct((L, D, B), jnp.float32),
        ],
    )(p_b, t, behT, padT, St)
    return (hp.transpose(2, 0, 1), hn.transpose(2, 0, 1))
